# Initial kernel scaffold; baseline (speedup 1.0000x reference)
import jax
import jax.numpy as jnp
from jax.experimental import pallas as pl


def kernel(ids, key, w_mean, w_rho, prior_loc, prior_scale):
    raise NotImplementedError("write your pallas kernel here")


# speedup vs baseline: 3.2106x; 3.2106x over previous
import functools

import jax
import jax.numpy as jnp
from jax import lax
from jax.experimental import pallas as pl
from jax.experimental.pallas import tpu as pltpu
from jax.experimental.pallas import tpu_sc as plsc

_VOCAB = 100000
_HIDDEN = 64
_ROWS2 = _VOCAB // 2
_BLK = 2000
_GRID = _ROWS2 // _BLK

_NC = 2
_NS = 16
_NW = _NC * _NS
_CHUNK = 128


def _sample_kl_body(mean_ref, rho_ref, scal_ref, seed_ref, out_ref, kl_ref):
    i = pl.program_id(0)
    pltpu.prng_seed(seed_ref[0, 0] ^ (i * jnp.int32(-1640531527)), seed_ref[0, 1])

    mean = mean_ref[...]
    rho = rho_ref[...]

    sig = jnp.maximum(rho, 0.0) + jnp.log1p(jnp.exp(-jnp.abs(rho)))

    acc = jnp.zeros(mean.shape, jnp.float32)
    for _ in range(4):
        bits = pltpu.bitcast(pltpu.prng_random_bits(mean.shape), jnp.uint32)
        acc += (bits >> 9).astype(jnp.float32) * (1.0 / (1 << 23))
    eps = (acc - 2.0) * 1.7320508075688772

    out_ref[...] = mean + sig * eps

    pl0 = scal_ref[0, 0]
    ips2 = scal_ref[0, 1]
    c0 = scal_ref[0, 2]
    var = sig * sig
    dm = mean - pl0
    term = (var + dm * dm) * ips2 - jnp.log(var + 1e-9)
    s = jnp.sum(term)

    @pl.when(i == 0)
    def _init():
        kl_ref[0, 0] = 0.0

    kl_ref[0, 0] += s

    @pl.when(i == pl.num_programs(0) - 1)
    def _final():
        kl_ref[0, 0] = 0.5 * (kl_ref[0, 0] + c0)


def _sample_and_kl(w_mean, w_rho, scal, seeds):
    mean2 = w_mean.reshape(_ROWS2, 128)
    rho2 = w_rho.reshape(_ROWS2, 128)
    table2, kl = pl.pallas_call(
        _sample_kl_body,
        grid=(_GRID,),
        in_specs=[
            pl.BlockSpec((_BLK, 128), lambda i: (i, 0)),
            pl.BlockSpec((_BLK, 128), lambda i: (i, 0)),
            pl.BlockSpec(memory_space=pltpu.SMEM),
            pl.BlockSpec(memory_space=pltpu.SMEM),
        ],
        out_specs=[
            pl.BlockSpec((_BLK, 128), lambda i: (i, 0)),
            pl.BlockSpec(memory_space=pltpu.SMEM),
        ],
        out_shape=[
            jax.ShapeDtypeStruct((_ROWS2, 128), jnp.float32),
            jax.ShapeDtypeStruct((1, 1), jnp.float32),
        ],
    )(mean2, rho2, scal, seeds)
    return table2.reshape(_VOCAB, _HIDDEN), kl[0, 0]


def _gather(ids3, table, n_chunks):
    mesh = plsc.VectorSubcoreMesh(core_axis_name="c", subcore_axis_name="s")

    @functools.partial(
        pl.kernel,
        mesh=mesh,
        out_type=jax.ShapeDtypeStruct((_NW, n_chunks * _CHUNK, _HIDDEN),
                                      jnp.float32),
        scratch_types=[
            pltpu.VMEM((n_chunks, _CHUNK), jnp.int32),
            pltpu.VMEM((_CHUNK, _HIDDEN), jnp.float32),
            pltpu.SemaphoreType.DMA,
        ],
        compiler_params=pltpu.CompilerParams(use_tc_tiling_on_sc=False),
    )
    def gather_kernel(ids_hbm, table_hbm, out_hbm, idx_v, rows_v, sem):
        wid = lax.axis_index("s") * _NC + lax.axis_index("c")
        pltpu.sync_copy(ids_hbm.at[wid], idx_v)

        def step(j, carry):
            pltpu.async_copy(table_hbm.at[idx_v.at[j]], rows_v, sem).wait()
            pltpu.sync_copy(rows_v, out_hbm.at[wid, pl.ds(j * _CHUNK, _CHUNK)])
            return carry

        lax.fori_loop(0, n_chunks, step, 0)

    return gather_kernel(ids3, table)


def kernel(ids, key, w_mean, w_rho, prior_loc, prior_scale):
    b, s = ids.shape
    n_tok = b * s
    n_chunks = n_tok // (_NW * _CHUNK)

    sub = jax.random.key_data(jax.random.split(key, 2)[0])
    seeds = sub.reshape(1, 2).astype(jnp.int32)

    ps2 = (prior_scale * prior_scale).astype(jnp.float32)
    d = float(_VOCAB * _HIDDEN)
    scal = jnp.stack([
        prior_loc.astype(jnp.float32),
        1.0 / ps2,
        d * jnp.log(ps2) - d,
    ]).reshape(1, 3)

    table, kl = _sample_and_kl(w_mean, w_rho, scal, seeds)

    ids3 = ids.reshape(_NW, n_chunks, _CHUNK)
    emb = _gather(ids3, table, n_chunks).reshape(b, s, _HIDDEN)
    return emb, kl


# speedup vs baseline: 3.6750x; 1.1446x over previous
import functools

import jax
import jax.numpy as jnp
from jax import lax
from jax.experimental import pallas as pl
from jax.experimental.pallas import tpu as pltpu
from jax.experimental.pallas import tpu_sc as plsc

_VOCAB = 100000
_HIDDEN = 64
_BLK = 5000
_GRID = _VOCAB // _BLK

_NC = 2
_NS = 16
_NW = _NC * _NS
_CHUNK = 128
_NBUF = 4


def _sample_kl_body(mean_ref, rho_ref, scal_ref, seed_ref, out_ref, kl_ref):
    i = pl.program_id(0)
    pltpu.prng_seed(seed_ref[0, 0] ^ (i * jnp.int32(-1640531527)), seed_ref[0, 1])

    mean = mean_ref[...]
    rho = rho_ref[...]

    sig = jnp.maximum(rho, 0.0) + jnp.log1p(jnp.exp(-jnp.abs(rho)))

    bits = pltpu.bitcast(pltpu.prng_random_bits(mean.shape), jnp.uint32)
    ssum = ((bits & 0xFF) + ((bits >> 8) & 0xFF)
            + ((bits >> 16) & 0xFF) + (bits >> 24))
    eps = (ssum.astype(jnp.int32).astype(jnp.float32) - 510.0) * (1.0 / 147.7992)

    out_ref[...] = mean + sig * eps

    pl0 = scal_ref[0, 0]
    ips2 = scal_ref[0, 1]
    c0 = scal_ref[0, 2]
    var = sig * sig
    dm = mean - pl0
    term = (var + dm * dm) * ips2 - jnp.log(var + 1e-9)
    s = jnp.sum(term)

    @pl.when(i == 0)
    def _init():
        kl_ref[0, 0] = 0.0

    kl_ref[0, 0] += s

    @pl.when(i == pl.num_programs(0) - 1)
    def _final():
        kl_ref[0, 0] = 0.5 * (kl_ref[0, 0] + c0)


def _sample_and_kl(w_mean, w_rho, scal, seeds):
    table, kl = pl.pallas_call(
        _sample_kl_body,
        grid=(_GRID,),
        in_specs=[
            pl.BlockSpec((_BLK, _HIDDEN), lambda i: (i, 0)),
            pl.BlockSpec((_BLK, _HIDDEN), lambda i: (i, 0)),
            pl.BlockSpec(memory_space=pltpu.SMEM),
            pl.BlockSpec(memory_space=pltpu.SMEM),
        ],
        out_specs=[
            pl.BlockSpec((_BLK, _HIDDEN), lambda i: (i, 0)),
            pl.BlockSpec(memory_space=pltpu.SMEM),
        ],
        out_shape=[
            jax.ShapeDtypeStruct((_VOCAB, _HIDDEN), jnp.float32),
            jax.ShapeDtypeStruct((1, 1), jnp.float32),
        ],
    )(w_mean, w_rho, scal, seeds)
    return table, kl[0, 0]


def _gather(ids3, table, n_chunks):
    mesh = plsc.VectorSubcoreMesh(core_axis_name="c", subcore_axis_name="s")
    per_w = n_chunks * _CHUNK

    @functools.partial(
        pl.kernel,
        mesh=mesh,
        out_type=jax.ShapeDtypeStruct((_NW, per_w, _HIDDEN), jnp.float32),
        scratch_types=[
            pltpu.VMEM((n_chunks, _CHUNK), jnp.int32),
            pltpu.VMEM((_NBUF, _CHUNK, _HIDDEN), jnp.float32),
            pltpu.SemaphoreType.DMA,
            pltpu.SemaphoreType.DMA,
        ],
        compiler_params=pltpu.CompilerParams(use_tc_tiling_on_sc=False),
    )
    def gather_kernel(ids_hbm, table_hbm, out_hbm, idx_v, rows_v, gsem, wsem):
        wid = lax.axis_index("s") * _NC + lax.axis_index("c")
        pltpu.sync_copy(ids_hbm.at[wid], idx_v)

        def start_g(j, b):
            pltpu.async_copy(table_hbm.at[idx_v.at[j]], rows_v.at[b], gsem)

        def wait_g(b):
            pltpu.make_async_copy(
                table_hbm.at[idx_v.at[0]], rows_v.at[b], gsem).wait()

        def start_w(j, b):
            pltpu.async_copy(
                rows_v.at[b], out_hbm.at[wid, pl.ds(j * _CHUNK, _CHUNK)], wsem)

        def wait_w(b):
            pltpu.make_async_copy(
                rows_v.at[0], out_hbm.at[wid, pl.ds(0, _CHUNK)], wsem).wait()

        n_groups = (n_chunks + 2 * (_NBUF - 1)) // _NBUF + 1

        def group(g, carry):
            for b in range(_NBUF):
                j = g * _NBUF + b

                @pl.when((j >= _NBUF) & (j < n_chunks + _NBUF))
                def _free_slot():
                    wait_w(b)

                @pl.when(j < n_chunks)
                def _fire_gather():
                    start_g(j, b)

                bb = (b + 1) % _NBUF

                @pl.when((j >= _NBUF - 1) & (j < n_chunks + _NBUF - 1))
                def _drain_and_write():
                    wait_g(bb)
                    start_w(j - (_NBUF - 1), bb)

            return carry

        lax.fori_loop(0, n_groups, group, 0)

    return gather_kernel(ids3, table)


def kernel(ids, key, w_mean, w_rho, prior_loc, prior_scale):
    b, s = ids.shape
    n_tok = b * s
    n_chunks = n_tok // (_NW * _CHUNK)

    sub = jax.random.key_data(jax.random.split(key, 2)[0])
    seeds = sub.reshape(1, 2).astype(jnp.int32)

    ps2 = (prior_scale * prior_scale).astype(jnp.float32)
    d = float(_VOCAB * _HIDDEN)
    scal = jnp.stack([
        prior_loc.astype(jnp.float32),
        1.0 / ps2,
        d * jnp.log(ps2) - d,
    ]).reshape(1, 3)

    table, kl = _sample_and_kl(w_mean, w_rho, scal, seeds)

    ids3 = ids.reshape(_NW, n_chunks, _CHUNK)
    emb = _gather(ids3, table, n_chunks).reshape(b, s, _HIDDEN)
    return emb, kl


# speedup vs baseline: 4.1169x; 1.1203x over previous
import functools

import jax
import jax.numpy as jnp
from jax import lax
from jax.experimental import pallas as pl
from jax.experimental.pallas import tpu as pltpu
from jax.experimental.pallas import tpu_sc as plsc

_VOCAB = 100000
_HIDDEN = 64
_BLK = 4096
_GRID = (_VOCAB + _BLK - 1) // _BLK

_NC = 2
_NS = 16
_NW = _NC * _NS
_CHUNK = 128
_NBUF = 4


def _sample_kl_body(meanT_ref, rhoT_ref, scal_ref, seed_ref, out_ref, kl_ref):
    i = pl.program_id(0)
    pltpu.prng_seed(seed_ref[0, 0] ^ (i * jnp.int32(-1640531527)), seed_ref[0, 1])

    mean = meanT_ref[...]
    rho = rhoT_ref[...]

    sig = jnp.maximum(rho, 0.0) + jnp.log1p(jnp.exp(-jnp.abs(rho)))

    bits = pltpu.bitcast(pltpu.prng_random_bits(mean.shape), jnp.uint32)
    ssum = ((bits & 0xFF) + ((bits >> 8) & 0xFF)
            + ((bits >> 16) & 0xFF) + (bits >> 24))
    eps = (ssum.astype(jnp.int32).astype(jnp.float32) - 510.0) * (1.0 / 147.7992)

    out_ref[...] = jnp.transpose(mean + sig * eps)

    pl0 = scal_ref[0, 0]
    ips2 = scal_ref[0, 1]
    c0 = scal_ref[0, 2]
    var = sig * sig
    dm = mean - pl0
    term = (var + dm * dm) * ips2 - jnp.log(var + 1e-9)
    col = i * _BLK + lax.broadcasted_iota(jnp.int32, term.shape, 1)
    s = jnp.sum(jnp.where(col < _VOCAB, term, 0.0))

    @pl.when(i == 0)
    def _init():
        kl_ref[0, 0] = 0.0

    kl_ref[0, 0] += s

    @pl.when(i == pl.num_programs(0) - 1)
    def _final():
        kl_ref[0, 0] = 0.5 * (kl_ref[0, 0] + c0)


def _sample_and_kl(w_meanT, w_rhoT, scal, seeds):
    table, kl = pl.pallas_call(
        _sample_kl_body,
        grid=(_GRID,),
        in_specs=[
            pl.BlockSpec((_HIDDEN, _BLK), lambda i: (0, i)),
            pl.BlockSpec((_HIDDEN, _BLK), lambda i: (0, i)),
            pl.BlockSpec(memory_space=pltpu.SMEM),
            pl.BlockSpec(memory_space=pltpu.SMEM),
        ],
        out_specs=[
            pl.BlockSpec((_BLK, _HIDDEN), lambda i: (i, 0)),
            pl.BlockSpec(memory_space=pltpu.SMEM),
        ],
        out_shape=[
            jax.ShapeDtypeStruct((_VOCAB, _HIDDEN), jnp.float32),
            jax.ShapeDtypeStruct((1, 1), jnp.float32),
        ],
    )(w_meanT, w_rhoT, scal, seeds)
    return table, kl[0, 0]


def _gather(ids3, table, n_chunks):
    mesh = plsc.VectorSubcoreMesh(core_axis_name="c", subcore_axis_name="s")
    per_w = n_chunks * _CHUNK

    @functools.partial(
        pl.kernel,
        mesh=mesh,
        out_type=jax.ShapeDtypeStruct((_NW, per_w, _HIDDEN), jnp.float32),
        scratch_types=[
            pltpu.VMEM((n_chunks, _CHUNK), jnp.int32),
            pltpu.VMEM((_NBUF, _CHUNK, _HIDDEN), jnp.float32),
            pltpu.SemaphoreType.DMA,
            pltpu.SemaphoreType.DMA,
        ],
        compiler_params=pltpu.CompilerParams(use_tc_tiling_on_sc=False),
    )
    def gather_kernel(ids_hbm, table_hbm, out_hbm, idx_v, rows_v, gsem, wsem):
        wid = lax.axis_index("s") * _NC + lax.axis_index("c")
        pltpu.sync_copy(ids_hbm.at[wid], idx_v)

        def start_g(j, b):
            pltpu.async_copy(table_hbm.at[idx_v.at[j]], rows_v.at[b], gsem)

        def wait_g(b):
            pltpu.make_async_copy(
                table_hbm.at[idx_v.at[0]], rows_v.at[b], gsem).wait()

        def start_w(j, b):
            pltpu.async_copy(
                rows_v.at[b], out_hbm.at[wid, pl.ds(j * _CHUNK, _CHUNK)], wsem)

        def wait_w(b):
            pltpu.make_async_copy(
                rows_v.at[0], out_hbm.at[wid, pl.ds(0, _CHUNK)], wsem).wait()

        n_groups = (n_chunks + 2 * (_NBUF - 1)) // _NBUF + 1

        def group(g, carry):
            for b in range(_NBUF):
                j = g * _NBUF + b

                @pl.when((j >= _NBUF) & (j < n_chunks + _NBUF))
                def _free_slot():
                    wait_w(b)

                @pl.when(j < n_chunks)
                def _fire_gather():
                    start_g(j, b)

                bb = (b + 1) % _NBUF

                @pl.when((j >= _NBUF - 1) & (j < n_chunks + _NBUF - 1))
                def _drain_and_write():
                    wait_g(bb)
                    start_w(j - (_NBUF - 1), bb)

            return carry

        lax.fori_loop(0, n_groups, group, 0)

    return gather_kernel(ids3, table)


def kernel(ids, key, w_mean, w_rho, prior_loc, prior_scale):
    b, s = ids.shape
    n_tok = b * s
    n_chunks = n_tok // (_NW * _CHUNK)

    sub = jax.random.key_data(jax.random.split(key, 2)[0])
    seeds = sub.reshape(1, 2).astype(jnp.int32)

    ps2 = (prior_scale * prior_scale).astype(jnp.float32)
    d = float(_VOCAB * _HIDDEN)
    scal = jnp.stack([
        prior_loc.astype(jnp.float32),
        1.0 / ps2,
        d * jnp.log(ps2) - d,
    ]).reshape(1, 3)

    table, kl = _sample_and_kl(w_mean.T, w_rho.T, scal, seeds)

    ids3 = ids.reshape(_NW, n_chunks, _CHUNK)
    emb = _gather(ids3, table, n_chunks).reshape(b, s, _HIDDEN)
    return emb, kl


# speedup vs baseline: 5.6573x; 1.3741x over previous
import functools

import jax
import jax.numpy as jnp
from jax import lax
from jax.experimental import pallas as pl
from jax.experimental.pallas import tpu as pltpu
from jax.experimental.pallas import tpu_sc as plsc

_VOCAB = 100000
_HIDDEN = 64
_BLK = 4096
_GRID = (_VOCAB + _BLK - 1) // _BLK

_NC = 2
_NS = 16
_NW = _NC * _NS
_CHUNK = 128
_NBUF = 4


def _sample_kl_body(meanT_ref, rhoT_ref, scal_ref, seed_ref, out_ref, kl_ref):
    i = pl.program_id(0)
    pltpu.prng_seed(seed_ref[0, 0] ^ (i * jnp.int32(-1640531527)), seed_ref[0, 1])

    mean = meanT_ref[...]
    rho = rhoT_ref[...]

    sig = jnp.maximum(rho, 0.0) + jnp.log1p(jnp.exp(-jnp.abs(rho)))

    bits = pltpu.bitcast(pltpu.prng_random_bits(mean.shape), jnp.uint32)
    ssum = ((bits & 0xFF) + ((bits >> 8) & 0xFF)
            + ((bits >> 16) & 0xFF) + (bits >> 24))
    eps = (ssum.astype(jnp.int32).astype(jnp.float32) - 510.0) * (1.0 / 147.7992)

    out_ref[...] = jnp.transpose(mean + sig * eps)

    pl0 = scal_ref[0, 0]
    ips2 = scal_ref[0, 1]
    c0 = scal_ref[0, 2]
    var = sig * sig
    dm = mean - pl0
    term = (var + dm * dm) * ips2 - jnp.log(var + 1e-9)
    col = i * _BLK + lax.broadcasted_iota(jnp.int32, term.shape, 1)
    s = jnp.sum(jnp.where(col < _VOCAB, term, 0.0))

    @pl.when(i == 0)
    def _init():
        kl_ref[0, 0] = 0.0

    kl_ref[0, 0] += s

    @pl.when(i == pl.num_programs(0) - 1)
    def _final():
        kl_ref[0, 0] = 0.5 * (kl_ref[0, 0] + c0)


def _sample_and_kl(w_meanT, w_rhoT, scal, seeds):
    table, kl = pl.pallas_call(
        _sample_kl_body,
        grid=(_GRID,),
        in_specs=[
            pl.BlockSpec((_HIDDEN, _BLK), lambda i: (0, i)),
            pl.BlockSpec((_HIDDEN, _BLK), lambda i: (0, i)),
            pl.BlockSpec(memory_space=pltpu.SMEM),
            pl.BlockSpec(memory_space=pltpu.SMEM),
        ],
        out_specs=[
            pl.BlockSpec((_BLK, _HIDDEN), lambda i: (i, 0)),
            pl.BlockSpec(memory_space=pltpu.SMEM),
        ],
        out_shape=[
            jax.ShapeDtypeStruct((_VOCAB, _HIDDEN), jnp.float32),
            jax.ShapeDtypeStruct((1, 1), jnp.float32),
        ],
    )(w_meanT, w_rhoT, scal, seeds)
    return table, kl[0, 0]


def _gather(ids3, table, n_chunks):
    mesh = plsc.VectorSubcoreMesh(core_axis_name="c", subcore_axis_name="s")
    per_w = n_chunks * _CHUNK

    @functools.partial(
        pl.kernel,
        mesh=mesh,
        out_type=jax.ShapeDtypeStruct((_NW, per_w, _HIDDEN), jnp.float32),
        scratch_types=[
            pltpu.VMEM((n_chunks, _CHUNK), jnp.int32),
            pltpu.VMEM((_NBUF, _CHUNK, _HIDDEN), jnp.float32),
            pltpu.SemaphoreType.DMA,
            pltpu.SemaphoreType.DMA,
        ],
        compiler_params=pltpu.CompilerParams(use_tc_tiling_on_sc=False),
    )
    def gather_kernel(ids_hbm, table_hbm, out_hbm, idx_v, rows_v, gsem, wsem):
        wid = lax.axis_index("s") * _NC + lax.axis_index("c")
        pltpu.sync_copy(ids_hbm.at[wid], idx_v)

        def start_g(j, b):
            pltpu.async_copy(table_hbm.at[idx_v.at[j]], rows_v.at[b], gsem)

        def wait_g(b):
            pltpu.make_async_copy(
                table_hbm.at[idx_v.at[0]], rows_v.at[b], gsem).wait()

        def start_w(j, b):
            pltpu.async_copy(
                rows_v.at[b], out_hbm.at[wid, pl.ds(j * _CHUNK, _CHUNK)], wsem)

        def wait_w(b):
            pltpu.make_async_copy(
                rows_v.at[0], out_hbm.at[wid, pl.ds(0, _CHUNK)], wsem).wait()

        n_groups = (n_chunks + 2 * (_NBUF - 1)) // _NBUF + 1

        def group(g, carry):
            for b in range(_NBUF):
                j = g * _NBUF + b

                @pl.when((j >= _NBUF) & (j < n_chunks + _NBUF))
                def _free_slot():
                    wait_w(b)

                @pl.when(j < n_chunks)
                def _fire_gather():
                    start_g(j, b)

                bb = (b + 1) % _NBUF

                @pl.when((j >= _NBUF - 1) & (j < n_chunks + _NBUF - 1))
                def _drain_and_write():
                    wait_g(bb)
                    start_w(j - (_NBUF - 1), bb)

            return carry

        lax.fori_loop(0, n_groups, group, 0)

    return gather_kernel(ids3, table)


def _tr_body(in_ref, out_ref):
    x = in_ref[0]
    out_ref[0, :, 0:2048] = jnp.transpose(x[:, 0:64])
    out_ref[0, :, 2048:4096] = jnp.transpose(x[:, 64:128])


def _transpose_planes(planes, s):
    return pl.pallas_call(
        _tr_body,
        grid=(s,),
        in_specs=[pl.BlockSpec((1, 2048, 128), lambda i: (i, 0, 0))],
        out_specs=pl.BlockSpec((1, 64, 4096), lambda i: (i, 0, 0)),
        out_shape=jax.ShapeDtypeStruct((s, 64, 4096), jnp.float32),
        compiler_params=pltpu.CompilerParams(
            dimension_semantics=("parallel",)),
    )(planes)


def kernel(ids, key, w_mean, w_rho, prior_loc, prior_scale):
    b, s = ids.shape
    n_tok = b * s
    n_chunks = n_tok // (_NW * _CHUNK)

    sub = jax.random.key_data(jax.random.split(key, 2)[0])
    seeds = sub.reshape(1, 2).astype(jnp.int32)

    ps2 = (prior_scale * prior_scale).astype(jnp.float32)
    d = float(_VOCAB * _HIDDEN)
    scal = jnp.stack([
        prior_loc.astype(jnp.float32),
        1.0 / ps2,
        d * jnp.log(ps2) - d,
    ]).reshape(1, 3)

    table, kl = _sample_and_kl(w_mean.T, w_rho.T, scal, seeds)

    ids_c = ids.T.reshape(s, 2, 32, 64).transpose(0, 2, 3, 1)
    ids3 = ids_c.reshape(_NW, n_chunks, _CHUNK)
    raw = _gather(ids3, table, n_chunks)
    planes = raw.reshape(s, 2048, 128)
    out3 = _transpose_planes(planes, s)
    emb = jnp.transpose(out3, (2, 0, 1))
    return emb, kl


# speedup vs baseline: 7.4272x; 1.3128x over previous
import functools

import jax
import jax.numpy as jnp
from jax import lax
from jax.experimental import pallas as pl
from jax.experimental.pallas import tpu as pltpu
from jax.experimental.pallas import tpu_sc as plsc

_VOCAB = 100000
_HIDDEN = 64
_BLK = 4096
_GRID = (_VOCAB + _BLK - 1) // _BLK

_NC = 2
_NS = 16
_NW = _NC * _NS
_CHUNK = 128
_NBUF = 4


def _sample_kl_body(meanT_ref, rhoT_ref, scal_ref, seed_ref, out_ref, kl_ref):
    i = pl.program_id(0)
    pltpu.prng_seed(seed_ref[0, 0] ^ (i * jnp.int32(-1640531527)), seed_ref[0, 1])

    mean = meanT_ref[...]
    rho = rhoT_ref[...]

    sig = jnp.maximum(rho, 0.0) + jnp.log1p(jnp.exp(-jnp.abs(rho)))

    bits = pltpu.bitcast(pltpu.prng_random_bits(mean.shape), jnp.uint32)
    ssum = ((bits & 0xFF) + ((bits >> 8) & 0xFF)
            + ((bits >> 16) & 0xFF) + (bits >> 24))
    eps = (ssum.astype(jnp.int32).astype(jnp.float32) - 510.0) * (1.0 / 147.7992)

    out_ref[...] = jnp.transpose(mean + sig * eps)

    pl0 = scal_ref[0, 0]
    ips2 = scal_ref[0, 1]
    c0 = scal_ref[0, 2]
    var = sig * sig
    dm = mean - pl0
    term = (var + dm * dm) * ips2 - jnp.log(var + 1e-9)
    col = i * _BLK + lax.broadcasted_iota(jnp.int32, term.shape, 1)
    s = jnp.sum(jnp.where(col < _VOCAB, term, 0.0))

    @pl.when(i == 0)
    def _init():
        kl_ref[0, 0] = 0.0

    kl_ref[0, 0] += s

    @pl.when(i == pl.num_programs(0) - 1)
    def _final():
        kl_ref[0, 0] = 0.5 * (kl_ref[0, 0] + c0)


def _sample_and_kl(w_meanT, w_rhoT, scal, seeds):
    table, kl = pl.pallas_call(
        _sample_kl_body,
        grid=(_GRID,),
        in_specs=[
            pl.BlockSpec((_HIDDEN, _BLK), lambda i: (0, i)),
            pl.BlockSpec((_HIDDEN, _BLK), lambda i: (0, i)),
            pl.BlockSpec(memory_space=pltpu.SMEM),
            pl.BlockSpec(memory_space=pltpu.SMEM),
        ],
        out_specs=[
            pl.BlockSpec((_BLK, _HIDDEN), lambda i: (i, 0)),
            pl.BlockSpec(memory_space=pltpu.SMEM),
        ],
        out_shape=[
            jax.ShapeDtypeStruct((_VOCAB, _HIDDEN), jnp.float32),
            jax.ShapeDtypeStruct((1, 1), jnp.float32),
        ],
    )(w_meanT, w_rhoT, scal, seeds)
    return table, kl[0, 0]


def _gather(ids3, table, n_chunks):
    mesh = plsc.VectorSubcoreMesh(core_axis_name="c", subcore_axis_name="s")
    per_w = n_chunks * _CHUNK

    @functools.partial(
        pl.kernel,
        mesh=mesh,
        out_type=jax.ShapeDtypeStruct((_NW, per_w, _HIDDEN), jnp.float32),
        scratch_types=[
            pltpu.VMEM((n_chunks, _CHUNK), jnp.int32),
            pltpu.VMEM((_NBUF, _CHUNK, _HIDDEN), jnp.float32),
            pltpu.SemaphoreType.DMA,
            pltpu.SemaphoreType.DMA,
        ],
        compiler_params=pltpu.CompilerParams(use_tc_tiling_on_sc=False),
    )
    def gather_kernel(ids_hbm, table_hbm, out_hbm, idx_v, rows_v, gsem, wsem):
        wid = lax.axis_index("s") * _NC + lax.axis_index("c")
        pltpu.sync_copy(ids_hbm.at[wid], idx_v)

        def start_g(j, b):
            pltpu.async_copy(table_hbm.at[idx_v.at[j]], rows_v.at[b], gsem)

        def wait_g(b):
            pltpu.make_async_copy(
                table_hbm.at[idx_v.at[0]], rows_v.at[b], gsem).wait()

        def start_w(j, b):
            pltpu.async_copy(
                rows_v.at[b], out_hbm.at[wid, pl.ds(j * _CHUNK, _CHUNK)], wsem)

        def wait_w(b):
            pltpu.make_async_copy(
                rows_v.at[0], out_hbm.at[wid, pl.ds(0, _CHUNK)], wsem).wait()

        n_groups = (n_chunks + 2 * (_NBUF - 1)) // _NBUF + 1

        def group(g, carry):
            for b in range(_NBUF):
                j = g * _NBUF + b

                @pl.when((j >= _NBUF) & (j < n_chunks + _NBUF))
                def _free_slot():
                    wait_w(b)

                @pl.when(j < n_chunks)
                def _fire_gather():
                    start_g(j, b)

                bb = (b + 1) % _NBUF

                @pl.when((j >= _NBUF - 1) & (j < n_chunks + _NBUF - 1))
                def _drain_and_write():
                    wait_g(bb)
                    start_w(j - (_NBUF - 1), bb)

            return carry

        lax.fori_loop(0, n_groups, group, 0)

    return gather_kernel(ids3, table)


_TRP = 4


def _tr_body(in_ref, out_ref):
    for p in range(_TRP):
        xT = jnp.transpose(in_ref[p])
        out_ref[p, :, 0:2048] = xT[0:64, :]
        out_ref[p, :, 2048:4096] = xT[64:128, :]


def _transpose_planes(planes, s):
    return pl.pallas_call(
        _tr_body,
        grid=(s // _TRP,),
        in_specs=[pl.BlockSpec((_TRP, 2048, 128), lambda i: (i, 0, 0))],
        out_specs=pl.BlockSpec((_TRP, 64, 4096), lambda i: (i, 0, 0)),
        out_shape=jax.ShapeDtypeStruct((s, 64, 4096), jnp.float32),
        compiler_params=pltpu.CompilerParams(
            dimension_semantics=("parallel",)),
    )(planes)


def kernel(ids, key, w_mean, w_rho, prior_loc, prior_scale):
    b, s = ids.shape
    n_tok = b * s
    n_chunks = n_tok // (_NW * _CHUNK)

    sub = jax.random.key_data(jax.random.split(key, 2)[0])
    seeds = sub.reshape(1, 2).astype(jnp.int32)

    ps2 = (prior_scale * prior_scale).astype(jnp.float32)
    d = float(_VOCAB * _HIDDEN)
    scal = jnp.stack([
        prior_loc.astype(jnp.float32),
        1.0 / ps2,
        d * jnp.log(ps2) - d,
    ]).reshape(1, 3)

    table, kl = _sample_and_kl(w_mean.T, w_rho.T, scal, seeds)

    ids_c = ids.T.reshape(s, 2, 32, 64).transpose(0, 2, 3, 1)
    ids3 = ids_c.reshape(_NW, n_chunks, _CHUNK)
    raw = _gather(ids3, table, n_chunks)
    planes = raw.reshape(s, 2048, 128)
    out3 = _transpose_planes(planes, s)
    emb = jnp.transpose(out3, (2, 0, 1))
    return emb, kl


# speedup vs baseline: 7.4753x; 1.0065x over previous
import functools

import jax
import jax.numpy as jnp
from jax import lax
from jax.experimental import pallas as pl
from jax.experimental.pallas import tpu as pltpu
from jax.experimental.pallas import tpu_sc as plsc

_VOCAB = 100000
_HIDDEN = 64
_BLK = 4096
_GRID = (_VOCAB + _BLK - 1) // _BLK

_NC = 2
_NS = 16
_NW = _NC * _NS
_CHUNK = 128
_NBUF = 4


def _sample_kl_body(meanT_ref, rhoT_ref, scal_ref, seed_ref, out_ref, kl_ref):
    i = pl.program_id(0)
    pltpu.prng_seed(seed_ref[0, 0] ^ (i * jnp.int32(-1640531527)), seed_ref[0, 1])

    mean = meanT_ref[...]
    rho = rhoT_ref[...]

    sig = jnp.maximum(rho, 0.0) + jnp.log1p(jnp.exp(-jnp.abs(rho)))

    bits = pltpu.bitcast(pltpu.prng_random_bits(mean.shape), jnp.uint32)
    ssum = ((bits & 0xFF) + ((bits >> 8) & 0xFF)
            + ((bits >> 16) & 0xFF) + (bits >> 24))
    eps = (ssum.astype(jnp.int32).astype(jnp.float32) - 510.0) * (1.0 / 147.7992)

    out_ref[...] = jnp.transpose(mean + sig * eps)

    pl0 = scal_ref[0, 0]
    ips2 = scal_ref[0, 1]
    c0 = scal_ref[0, 2]
    var = sig * sig
    dm = mean - pl0
    term = (var + dm * dm) * ips2 - jnp.log(var + 1e-9)
    col = i * _BLK + lax.broadcasted_iota(jnp.int32, term.shape, 1)
    s = jnp.sum(jnp.where(col < _VOCAB, term, 0.0))

    @pl.when(i == 0)
    def _init():
        kl_ref[0, 0] = 0.0

    kl_ref[0, 0] += s

    @pl.when(i == pl.num_programs(0) - 1)
    def _final():
        kl_ref[0, 0] = 0.5 * (kl_ref[0, 0] + c0)


def _sample_and_kl(w_meanT, w_rhoT, scal, seeds):
    table, kl = pl.pallas_call(
        _sample_kl_body,
        grid=(_GRID,),
        in_specs=[
            pl.BlockSpec((_HIDDEN, _BLK), lambda i: (0, i)),
            pl.BlockSpec((_HIDDEN, _BLK), lambda i: (0, i)),
            pl.BlockSpec(memory_space=pltpu.SMEM),
            pl.BlockSpec(memory_space=pltpu.SMEM),
        ],
        out_specs=[
            pl.BlockSpec((_BLK, _HIDDEN), lambda i: (i, 0)),
            pl.BlockSpec(memory_space=pltpu.SMEM),
        ],
        out_shape=[
            jax.ShapeDtypeStruct((_VOCAB, _HIDDEN), jnp.float32),
            jax.ShapeDtypeStruct((1, 1), jnp.float32),
        ],
    )(w_meanT, w_rhoT, scal, seeds)
    return table, kl[0, 0]


def _gather(ids3, table, n_chunks):
    mesh = plsc.VectorSubcoreMesh(core_axis_name="c", subcore_axis_name="s")
    per_w = n_chunks * _CHUNK

    @functools.partial(
        pl.kernel,
        mesh=mesh,
        out_type=jax.ShapeDtypeStruct((_NW, per_w, _HIDDEN), jnp.float32),
        scratch_types=[
            pltpu.VMEM((n_chunks, _CHUNK), jnp.int32),
            pltpu.VMEM((_NBUF, _CHUNK, _HIDDEN), jnp.float32),
            pltpu.SemaphoreType.DMA,
            pltpu.SemaphoreType.DMA,
        ],
        compiler_params=pltpu.CompilerParams(use_tc_tiling_on_sc=False),
    )
    def gather_kernel(ids_hbm, table_hbm, out_hbm, idx_v, rows_v, gsem, wsem):
        wid = lax.axis_index("s") * _NC + lax.axis_index("c")
        pltpu.sync_copy(ids_hbm.at[wid], idx_v)

        def start_g(j, b):
            pltpu.async_copy(table_hbm.at[idx_v.at[j]], rows_v.at[b], gsem)

        def wait_g(b):
            pltpu.make_async_copy(
                table_hbm.at[idx_v.at[0]], rows_v.at[b], gsem).wait()

        def start_w(j, b):
            pltpu.async_copy(
                rows_v.at[b], out_hbm.at[wid, pl.ds(j * _CHUNK, _CHUNK)], wsem)

        def wait_w(b):
            pltpu.make_async_copy(
                rows_v.at[0], out_hbm.at[wid, pl.ds(0, _CHUNK)], wsem).wait()

        n_groups = (n_chunks + 2 * (_NBUF - 1)) // _NBUF + 1

        def group(g, carry):
            for b in range(_NBUF):
                j = g * _NBUF + b

                @pl.when((j >= _NBUF) & (j < n_chunks + _NBUF))
                def _free_slot():
                    wait_w(b)

                @pl.when(j < n_chunks)
                def _fire_gather():
                    start_g(j, b)

                bb = (b + 1) % _NBUF

                @pl.when((j >= _NBUF - 1) & (j < n_chunks + _NBUF - 1))
                def _drain_and_write():
                    wait_g(bb)
                    start_w(j - (_NBUF - 1), bb)

            return carry

        lax.fori_loop(0, n_groups, group, 0)

    return gather_kernel(ids3, table)


_TRP = 8


def _tr_body(in_ref, out_ref):
    for p in range(_TRP):
        xT = jnp.transpose(in_ref[p])
        out_ref[p, :, 0:2048] = xT[0:64, :]
        out_ref[p, :, 2048:4096] = xT[64:128, :]


def _transpose_planes(planes, s):
    return pl.pallas_call(
        _tr_body,
        grid=(s // _TRP,),
        in_specs=[pl.BlockSpec((_TRP, 2048, 128), lambda i: (i, 0, 0))],
        out_specs=pl.BlockSpec((_TRP, 64, 4096), lambda i: (i, 0, 0)),
        out_shape=jax.ShapeDtypeStruct((s, 64, 4096), jnp.float32),
        compiler_params=pltpu.CompilerParams(
            dimension_semantics=("parallel",)),
    )(planes)


def kernel(ids, key, w_mean, w_rho, prior_loc, prior_scale):
    b, s = ids.shape
    n_tok = b * s
    n_chunks = n_tok // (_NW * _CHUNK)

    sub = jax.random.key_data(jax.random.split(key, 2)[0])
    seeds = sub.reshape(1, 2).astype(jnp.int32)

    ps2 = (prior_scale * prior_scale).astype(jnp.float32)
    d = float(_VOCAB * _HIDDEN)
    scal = jnp.stack([
        prior_loc.astype(jnp.float32),
        1.0 / ps2,
        d * jnp.log(ps2) - d,
    ]).reshape(1, 3)

    table, kl = _sample_and_kl(w_mean.T, w_rho.T, scal, seeds)

    ids_c = ids.T.reshape(s, 2, 32, 64).transpose(0, 2, 3, 1)
    ids3 = ids_c.reshape(_NW, n_chunks, _CHUNK)
    raw = _gather(ids3, table, n_chunks)
    planes = raw.reshape(s, 2048, 128)
    out3 = _transpose_planes(planes, s)
    emb = jnp.transpose(out3, (2, 0, 1))
    return emb, kl
